# Initial kernel scaffold; baseline (speedup 1.0000x reference)
import jax
import jax.numpy as jnp
from jax.experimental import pallas as pl


def kernel(input, plogit):
    raise NotImplementedError("write your pallas kernel here")


# speedup vs baseline: 17.2637x; 17.2637x over previous
import jax
import jax.numpy as jnp
from jax.experimental import pallas as pl
from jax.experimental.pallas import tpu as pltpu

_SPREAD = 0.01
_NBITS = 32


def _decode_key(kcode):
    bits = jnp.where(kcode >= 0, kcode, ~kcode | jnp.int32(-(2**31)))
    return jax.lax.bitcast_convert_type(bits, jnp.float32)


def _select_relu_kernel(k_ref, p_ref, x_ref, o_ref):
    x = x_ref[...]
    s = jax.lax.bitcast_convert_type(x, jnp.int32)
    key = jnp.where(s >= 0, s, ~(s & jnp.int32(0x7FFFFFFF)))

    rows = x.shape[0]
    k_low = k_ref[0, 0]
    k_high = k_ref[0, 1]
    init = jnp.full((rows, 1), jnp.int32(-(2**31)), jnp.int32)

    def body(i, carry):
        plo, phi = carry
        bit = jnp.int32(1) << (jnp.int32(31) - i)
        tlo = plo + bit
        thi = phi + bit
        clo = jnp.sum((key < tlo).astype(jnp.int32), axis=1, keepdims=True)
        chi = jnp.sum((key < thi).astype(jnp.int32), axis=1, keepdims=True)
        plo = jnp.where(clo > k_low, plo, tlo)
        phi = jnp.where(chi > k_high, phi, thi)
        return plo, phi

    plo, phi = jax.lax.fori_loop(0, _NBITS, body, (init, init))

    x_low = _decode_key(plo)
    x_high = _decode_key(phi)
    p = p_ref[0, 0]
    r_low = jnp.maximum(x - x_low, 0.0)
    r_high = jnp.maximum(x - x_high, 0.0)
    o_ref[...] = r_low + (r_high - r_low) * p


def kernel(input, plogit):
    shape = input.shape
    if input.ndim > 2:
        rows = shape[0] * shape[1]
    else:
        rows = shape[0]
    x = input.reshape(rows, -1)
    L = x.shape[-1]

    p_val = jax.nn.sigmoid(plogit)[0].astype(jnp.float32)
    k_low = jnp.clip((L * (p_val - _SPREAD)).astype(jnp.int32), 0, L - 1)
    k_high = jnp.clip((L * (p_val + _SPREAD)).astype(jnp.int32), 0, L - 1)
    kk = jnp.stack([k_low, k_high]).reshape(1, 2)
    pp = p_val.reshape(1, 1)

    R = 8 if rows % 8 == 0 else 1
    out = pl.pallas_call(
        _select_relu_kernel,
        grid=(rows // R,),
        in_specs=[
            pl.BlockSpec(memory_space=pltpu.SMEM),
            pl.BlockSpec(memory_space=pltpu.SMEM),
            pl.BlockSpec((R, L), lambda i: (i, 0)),
        ],
        out_specs=pl.BlockSpec((R, L), lambda i: (i, 0)),
        out_shape=jax.ShapeDtypeStruct((rows, L), jnp.float32),
    )(kk, pp, x)
    return out.reshape(shape)


# speedup vs baseline: 25.5182x; 1.4781x over previous
import jax
import jax.numpy as jnp
from jax.experimental import pallas as pl
from jax.experimental.pallas import tpu as pltpu

_SPREAD = 0.01
_NBITS = 20


def _decode_key(kcode):
    bits = jnp.where(kcode >= 0, kcode, ~kcode | jnp.int32(-(2**31)))
    return jax.lax.bitcast_convert_type(bits, jnp.float32)


def _select_relu_kernel(k_ref, p_ref, x_ref, o_ref):
    x = x_ref[...]
    s = jax.lax.bitcast_convert_type(x, jnp.int32)
    key = s ^ (jax.lax.shift_right_arithmetic(s, 31) & jnp.int32(0x7FFFFFFF))

    rows = x.shape[0]
    k_low = k_ref[0, 0]
    k_high = k_ref[0, 1]
    init = jnp.full((rows, 1), jnp.int32(-(2**31)), jnp.int32)

    def body(i, carry):
        plo, phi = carry
        bit = jnp.int32(1) << (jnp.int32(31) - i)
        tlo = plo + bit
        thi = phi + bit
        clo = jnp.sum((key < tlo).astype(jnp.int32), axis=1, keepdims=True)
        chi = jnp.sum((key < thi).astype(jnp.int32), axis=1, keepdims=True)
        plo = jnp.where(clo > k_low, plo, tlo)
        phi = jnp.where(chi > k_high, phi, thi)
        return plo, phi

    plo, phi = jax.lax.fori_loop(0, _NBITS, body, (init, init))

    x_low = _decode_key(plo)
    x_high = _decode_key(phi)
    p = p_ref[0, 0]
    r_low = jnp.maximum(x - x_low, 0.0)
    r_high = jnp.maximum(x - x_high, 0.0)
    o_ref[...] = r_low + (r_high - r_low) * p


def kernel(input, plogit):
    shape = input.shape
    if input.ndim > 2:
        rows = shape[0] * shape[1]
    else:
        rows = shape[0]
    x = input.reshape(rows, -1)
    L = x.shape[-1]

    p_val = jax.nn.sigmoid(plogit)[0].astype(jnp.float32)
    k_low = jnp.clip((L * (p_val - _SPREAD)).astype(jnp.int32), 0, L - 1)
    k_high = jnp.clip((L * (p_val + _SPREAD)).astype(jnp.int32), 0, L - 1)
    kk = jnp.stack([k_low, k_high]).reshape(1, 2)
    pp = p_val.reshape(1, 1)

    R = 16 if rows % 16 == 0 else (8 if rows % 8 == 0 else 1)
    out = pl.pallas_call(
        _select_relu_kernel,
        grid=(rows // R,),
        compiler_params=pltpu.CompilerParams(
            dimension_semantics=("parallel",)),
        in_specs=[
            pl.BlockSpec(memory_space=pltpu.SMEM),
            pl.BlockSpec(memory_space=pltpu.SMEM),
            pl.BlockSpec((R, L), lambda i: (i, 0)),
        ],
        out_specs=pl.BlockSpec((R, L), lambda i: (i, 0)),
        out_shape=jax.ShapeDtypeStruct((rows, L), jnp.float32),
    )(kk, pp, x)
    return out.reshape(shape)


# speedup vs baseline: 26.9879x; 1.0576x over previous
import jax
import jax.numpy as jnp
from jax.experimental import pallas as pl
from jax.experimental.pallas import tpu as pltpu

_SPREAD = 0.01
_NBITS = 16


def _decode_key(kcode):
    bits = jnp.where(kcode >= 0, kcode, ~kcode | jnp.int32(-(2**31)))
    return jax.lax.bitcast_convert_type(bits, jnp.float32)


def _select_relu_kernel(k_ref, p_ref, x_ref, o_ref):
    x = x_ref[...]
    s = jax.lax.bitcast_convert_type(x, jnp.int32)
    key = s ^ (jax.lax.shift_right_arithmetic(s, 31) & jnp.int32(0x7FFFFFFF))

    rows = x.shape[0]
    k_low = k_ref[0, 0]
    k_high = k_ref[0, 1]
    init = jnp.full((rows, 1), jnp.int32(-(2**31)), jnp.int32)

    def body(i, carry):
        plo, phi = carry
        bit = jnp.int32(1) << (jnp.int32(31) - i)
        tlo = plo + bit
        thi = phi + bit
        clo = jnp.sum((key < tlo).astype(jnp.int32), axis=1, keepdims=True)
        chi = jnp.sum((key < thi).astype(jnp.int32), axis=1, keepdims=True)
        plo = jnp.where(clo > k_low, plo, tlo)
        phi = jnp.where(chi > k_high, phi, thi)
        return plo, phi

    plo, phi = jax.lax.fori_loop(0, _NBITS, body, (init, init))

    q = jnp.int32(1) << jnp.int32(32 - _NBITS)
    L = jnp.int32(x.shape[1])

    def refine(pfx, k):
        top = pfx + q
        c0 = jnp.sum((key < pfx).astype(jnp.int32), axis=1, keepdims=True)
        c1 = jnp.where(top < pfx, L,
                       jnp.sum((key < top).astype(jnp.int32), axis=1,
                               keepdims=True))
        j = (k - c0 + 1).astype(jnp.float32)
        n1 = (c1 - c0 + 1).astype(jnp.float32)
        offs = (q.astype(jnp.float32) * (j / n1)).astype(jnp.int32)
        return jnp.where(top < pfx, pfx, pfx + offs)

    x_low = _decode_key(refine(plo, k_low))
    x_high = _decode_key(refine(phi, k_high))
    p = p_ref[0, 0]
    r_low = jnp.maximum(x - x_low, 0.0)
    r_high = jnp.maximum(x - x_high, 0.0)
    o_ref[...] = r_low + (r_high - r_low) * p


def kernel(input, plogit):
    shape = input.shape
    if input.ndim > 2:
        rows = shape[0] * shape[1]
    else:
        rows = shape[0]
    x = input.reshape(rows, -1)
    L = x.shape[-1]

    p_val = jax.nn.sigmoid(plogit)[0].astype(jnp.float32)
    k_low = jnp.clip((L * (p_val - _SPREAD)).astype(jnp.int32), 0, L - 1)
    k_high = jnp.clip((L * (p_val + _SPREAD)).astype(jnp.int32), 0, L - 1)
    kk = jnp.stack([k_low, k_high]).reshape(1, 2)
    pp = p_val.reshape(1, 1)

    R = 16 if rows % 16 == 0 else (8 if rows % 8 == 0 else 1)
    out = pl.pallas_call(
        _select_relu_kernel,
        grid=(rows // R,),
        compiler_params=pltpu.CompilerParams(
            dimension_semantics=("parallel",)),
        in_specs=[
            pl.BlockSpec(memory_space=pltpu.SMEM),
            pl.BlockSpec(memory_space=pltpu.SMEM),
            pl.BlockSpec((R, L), lambda i: (i, 0)),
        ],
        out_specs=pl.BlockSpec((R, L), lambda i: (i, 0)),
        out_shape=jax.ShapeDtypeStruct((rows, L), jnp.float32),
    )(kk, pp, x)
    return out.reshape(shape)


# speedup vs baseline: 27.0935x; 1.0039x over previous
import jax
import jax.numpy as jnp
from jax.experimental import pallas as pl
from jax.experimental.pallas import tpu as pltpu

_SPREAD = 0.01
_NBITS = 16


def _decode_key(kcode):
    bits = jnp.where(kcode >= 0, kcode, ~kcode | jnp.int32(-(2**31)))
    return jax.lax.bitcast_convert_type(bits, jnp.float32)


def _select_relu_kernel(k_ref, p_ref, x_ref, o_ref):
    x = x_ref[...]
    rows, L = x.shape
    s = jax.lax.bitcast_convert_type(x, jnp.int32)
    key32 = s ^ (jax.lax.shift_right_arithmetic(s, 31) & jnp.int32(0x7FFFFFFF))
    key = jax.lax.shift_right_arithmetic(key32, 16).astype(jnp.int16)

    if L % 1024 == 0:
        G = L // 1024
        key_c = key.reshape(rows, G, 8, 128)

        def count_lt(t):
            t16 = t.astype(jnp.int16)
            m = (key_c < t16[:, :, None, None]).astype(jnp.int16)
            parts = [m[:, g] for g in range(G)]
            while len(parts) > 1:
                nxt = [parts[a] + parts[a + 1]
                       for a in range(0, len(parts) - 1, 2)]
                if len(parts) % 2:
                    nxt.append(parts[-1])
                parts = nxt
            acc = parts[0].astype(jnp.int32).reshape(rows, 1024)
            return jnp.sum(acc, axis=1, keepdims=True)
    else:

        def count_lt(t):
            m = (key < t.astype(jnp.int16)).astype(jnp.int32)
            return jnp.sum(m, axis=1, keepdims=True)

    k_low = k_ref[0, 0]
    k_high = k_ref[0, 1]
    init = jnp.full((rows, 1), jnp.int32(-(2**15)), jnp.int32)

    def body(i, carry):
        plo, phi = carry
        bit = jnp.int32(1) << (jnp.int32(15) - i)
        tlo = plo + bit
        thi = phi + bit
        plo = jnp.where(count_lt(tlo) > k_low, plo, tlo)
        phi = jnp.where(count_lt(thi) > k_high, phi, thi)
        return plo, phi

    plo, phi = jax.lax.fori_loop(0, _NBITS, body, (init, init))

    Lc = jnp.int32(L)

    def refine(pfx, k):
        top = pfx + 1
        c0 = count_lt(pfx)
        c1 = jnp.where(top > 32767, Lc, count_lt(top))
        j = (k - c0 + 1).astype(jnp.float32)
        n1 = (c1 - c0 + 1).astype(jnp.float32)
        offs = (jnp.float32(65536.0) * (j / n1)).astype(jnp.int32)
        return (pfx << 16) + jnp.minimum(offs, 65535)

    x_low = _decode_key(refine(plo, k_low))
    x_high = _decode_key(refine(phi, k_high))
    p = p_ref[0, 0]
    r_low = jnp.maximum(x - x_low, 0.0)
    r_high = jnp.maximum(x - x_high, 0.0)
    o_ref[...] = r_low + (r_high - r_low) * p


def kernel(input, plogit):
    shape = input.shape
    if input.ndim > 2:
        rows = shape[0] * shape[1]
    else:
        rows = shape[0]
    x = input.reshape(rows, -1)
    L = x.shape[-1]

    p_val = jax.nn.sigmoid(plogit)[0].astype(jnp.float32)
    k_low = jnp.clip((L * (p_val - _SPREAD)).astype(jnp.int32), 0, L - 1)
    k_high = jnp.clip((L * (p_val + _SPREAD)).astype(jnp.int32), 0, L - 1)
    kk = jnp.stack([k_low, k_high]).reshape(1, 2)
    pp = p_val.reshape(1, 1)

    R = 16 if rows % 16 == 0 else (8 if rows % 8 == 0 else 1)
    out = pl.pallas_call(
        _select_relu_kernel,
        grid=(rows // R,),
        compiler_params=pltpu.CompilerParams(
            dimension_semantics=("parallel",)),
        in_specs=[
            pl.BlockSpec(memory_space=pltpu.SMEM),
            pl.BlockSpec(memory_space=pltpu.SMEM),
            pl.BlockSpec((R, L), lambda i: (i, 0)),
        ],
        out_specs=pl.BlockSpec((R, L), lambda i: (i, 0)),
        out_shape=jax.ShapeDtypeStruct((rows, L), jnp.float32),
    )(kk, pp, x)
    return out.reshape(shape)


# speedup vs baseline: 29.3673x; 1.0839x over previous
import jax
import jax.numpy as jnp
from jax.experimental import pallas as pl
from jax.experimental.pallas import tpu as pltpu

_SPREAD = 0.01
_NBITS = 16


def _decode_key(kcode):
    bits = jnp.where(kcode >= 0, kcode, ~kcode | jnp.int32(-(2**31)))
    return jax.lax.bitcast_convert_type(bits, jnp.float32)


def _select_relu_kernel(k_ref, p_ref, x_ref, o_ref):
    x = x_ref[...]
    rows, L = x.shape
    s = jax.lax.bitcast_convert_type(x, jnp.int32)
    key32 = s ^ (jax.lax.shift_right_arithmetic(s, 31) & jnp.int32(0x7FFFFFFF))
    key = jax.lax.shift_right_arithmetic(key32, 16).astype(jnp.int16)

    if L % 1024 == 0:
        G = L // 1024
        key_c = key.reshape(rows, G, 8, 128)

        def count2_lt(ta, tb):
            ta16 = ta.astype(jnp.int16)[:, :, None, None]
            tb16 = tb.astype(jnp.int16)[:, :, None, None]
            acc_a = jnp.zeros((rows, 8, 128), jnp.int16)
            acc_b = jnp.zeros((rows, 8, 128), jnp.int16)
            for g in range(G):
                kg = key_c[:, g]
                acc_a = acc_a + (kg < ta16[:, 0]).astype(jnp.int16)
                acc_b = acc_b + (kg < tb16[:, 0]).astype(jnp.int16)
            ca = jnp.sum(acc_a.astype(jnp.int32), axis=(1, 2))
            cb = jnp.sum(acc_b.astype(jnp.int32), axis=(1, 2))
            return ca[:, None], cb[:, None]
    else:

        def count2_lt(ta, tb):
            ca = jnp.sum((key < ta.astype(jnp.int16)).astype(jnp.int32),
                         axis=1, keepdims=True)
            cb = jnp.sum((key < tb.astype(jnp.int16)).astype(jnp.int32),
                         axis=1, keepdims=True)
            return ca, cb

    k_low = k_ref[0, 0]
    k_high = k_ref[0, 1]
    init = jnp.full((rows, 1), jnp.int32(-(2**15)), jnp.int32)

    def body(i, carry):
        plo, phi = carry
        bit = jnp.int32(1) << (jnp.int32(15) - i)
        tlo = plo + bit
        thi = phi + bit
        clo, chi = count2_lt(tlo, thi)
        plo = jnp.where(clo > k_low, plo, tlo)
        phi = jnp.where(chi > k_high, phi, thi)
        return plo, phi

    plo, phi = jax.lax.fori_loop(0, _NBITS, body, (init, init))

    Lc = jnp.int32(L)

    def refine(pfx, k):
        top = pfx + 1
        c0, c1 = count2_lt(pfx, top)
        c1 = jnp.where(top > 32767, Lc, c1)
        j = (k - c0 + 1).astype(jnp.float32)
        n1 = (c1 - c0 + 1).astype(jnp.float32)
        offs = (jnp.float32(65536.0) * (j / n1)).astype(jnp.int32)
        return (pfx << 16) + jnp.minimum(offs, 65535)

    x_low = _decode_key(refine(plo, k_low))
    x_high = _decode_key(refine(phi, k_high))
    p = p_ref[0, 0]
    r_low = jnp.maximum(x - x_low, 0.0)
    r_high = jnp.maximum(x - x_high, 0.0)
    o_ref[...] = r_low + (r_high - r_low) * p


def kernel(input, plogit):
    shape = input.shape
    if input.ndim > 2:
        rows = shape[0] * shape[1]
    else:
        rows = shape[0]
    x = input.reshape(rows, -1)
    L = x.shape[-1]

    p_val = jax.nn.sigmoid(plogit)[0].astype(jnp.float32)
    k_low = jnp.clip((L * (p_val - _SPREAD)).astype(jnp.int32), 0, L - 1)
    k_high = jnp.clip((L * (p_val + _SPREAD)).astype(jnp.int32), 0, L - 1)
    kk = jnp.stack([k_low, k_high]).reshape(1, 2)
    pp = p_val.reshape(1, 1)

    R = 16 if rows % 16 == 0 else (8 if rows % 8 == 0 else 1)
    out = pl.pallas_call(
        _select_relu_kernel,
        grid=(rows // R,),
        compiler_params=pltpu.CompilerParams(
            dimension_semantics=("parallel",)),
        in_specs=[
            pl.BlockSpec(memory_space=pltpu.SMEM),
            pl.BlockSpec(memory_space=pltpu.SMEM),
            pl.BlockSpec((R, L), lambda i: (i, 0)),
        ],
        out_specs=pl.BlockSpec((R, L), lambda i: (i, 0)),
        out_shape=jax.ShapeDtypeStruct((rows, L), jnp.float32),
    )(kk, pp, x)
    return out.reshape(shape)
